# baseline (speedup 1.0000x reference)
import functools

import jax
import jax.numpy as jnp
from jax import lax
from jax.experimental import pallas as pl
from jax.experimental.pallas import tpu as pltpu
from jax.experimental.pallas import tpu_sc as plsc

N = 10000
E = 320000
D = 128
HEADS = 4
NC = 2
NS = 16
NW = NC * NS
CHUNK = 128
NCHUNKS = E // CHUNK
RPT = 640
BLK = 1000


def _prep_body(h_ref, g_ref, b_ref, w_ref, a_ref,
               f0_ref, f1_ref, at0_ref, at1_ref, mcol_ref):
    nb = pl.program_id(0)

    @pl.when(nb == 0)
    def _():
        mcol_ref[...] = jnp.full((2, 16), -1e30, jnp.float32)

    h = h_ref[...]
    mu = jnp.mean(h, axis=1, keepdims=True)
    hc = h - mu
    var = jnp.mean(hc * hc, axis=1, keepdims=True)
    hn = hc * lax.rsqrt(var + 1e-5) * g_ref[...][None, :] + b_ref[...][None, :]

    f_refs = (f0_ref, f1_ref)
    at_refs = (at0_ref, at1_ref)
    for r in range(2):
        feat = jnp.dot(hn, w_ref[r], preferred_element_type=jnp.float32)
        at = jnp.dot(feat, a_ref[r], preferred_element_type=jnp.float32)
        f_refs[r][...] = feat
        at_refs[r][...] = at
        mcol_ref[r, :] = jnp.maximum(mcol_ref[r, :], jnp.max(at, axis=0))


def _prep(h, g, b, W, A):
    return pl.pallas_call(
        _prep_body,
        grid=(N // BLK,),
        in_specs=[
            pl.BlockSpec((BLK, D), lambda i: (i, 0)),
            pl.BlockSpec((D,), lambda i: (0,)),
            pl.BlockSpec((D,), lambda i: (0,)),
            pl.BlockSpec((2, D, D), lambda i: (0, 0, 0)),
            pl.BlockSpec((2, D, 16), lambda i: (0, 0, 0)),
        ],
        out_specs=[
            pl.BlockSpec((BLK, D), lambda i: (i, 0)),
            pl.BlockSpec((BLK, D), lambda i: (i, 0)),
            pl.BlockSpec((BLK, 16), lambda i: (i, 0)),
            pl.BlockSpec((BLK, 16), lambda i: (i, 0)),
            pl.BlockSpec((2, 16), lambda i: (0, 0)),
        ],
        out_shape=[
            jax.ShapeDtypeStruct((N, D), jnp.float32),
            jax.ShapeDtypeStruct((N, D), jnp.float32),
            jax.ShapeDtypeStruct((N, 16), jnp.float32),
            jax.ShapeDtypeStruct((N, 16), jnp.float32),
            jax.ShapeDtypeStruct((2, 16), jnp.float32),
        ],
    )(h, g, b, W, A)


def _dyn_gather(v, idx):
    dnums = lax.GatherDimensionNumbers(
        offset_dims=(), collapsed_slice_dims=(0,), start_index_map=(0,))
    return lax.gather(v, idx[:, None], dnums, (1,),
                      mode=lax.GatherScatterMode.PROMISE_IN_BOUNDS)


GCH = 80
NI = E // GCH // NW


def _sc_gat_body(feat_hbm, at_hbm, ei_hbm, mv_hbm,
                 num_hbm, den_hbm,
                 idx0, rows0, ats0, atd0, exv0,
                 idx1, rows1, ats1, atd1, exv1,
                 nidx0, nidx1,
                 mvv, snum, sden, gsem0, ssem0, gsem1, ssem1, isem0, isem1):
    cid = lax.axis_index("c")
    sid = lax.axis_index("s")
    wid = sid * NC + cid

    SL = [
        dict(idx=idx0, rows=rows0, ats=ats0, atd=atd0, exv=exv0,
             gsem=gsem0, ssem=ssem0),
        dict(idx=idx1, rows=rows1, ats=ats1, atd=atd1, exv=exv1,
             gsem=gsem1, ssem=ssem1),
    ]
    rows = rows0
    exv = exv0

    zero16 = jnp.zeros((16,), jnp.float32)

    def zbody(i, _):
        for j in range(D // 16):
            rows[i, pl.ds(j * 16, 16)] = zero16
        exv[i, :] = zero16
        return 0
    lax.fori_loop(0, GCH, zbody, 0)

    base = sid * RPT

    def _tile_rows(fn):
        @pl.when(sid < NS - 1)
        def _():
            for k in range(RPT // GCH):
                fn(base + k * GCH)

        @pl.when(sid == NS - 1)
        def _():
            for k in range((N - (NS - 1) * RPT) // GCH):
                fn(base + k * GCH)

    def _zero(off):
        pltpu.sync_copy(rows.at[pl.ds(0, GCH)], snum.at[pl.ds(off, GCH)])
        pltpu.sync_copy(exv.at[pl.ds(0, GCH)], sden.at[pl.ds(off, GCH)])
    _tile_rows(_zero)

    pltpu.sync_copy(mv_hbm, mvv)
    plsc.subcore_barrier()

    mvec = mvv[:]
    lane = lax.iota(jnp.int32, 16)
    pidx_l = lane % 4
    pidx_r = pidx_l + 4
    hsplat = [lane * 0 + h for h in range(HEADS)]

    isems = {id(nidx0): isem0, id(nidx1): isem1}

    def fetch_idx(nidx, it):
        base_e = (wid + it * NW) * GCH
        pltpu.async_copy(ei_hbm.at[:, pl.ds(base_e, GCH)], nidx, isems[id(nidx)])

    def load(S, nidx):
        for r in range(2):
            for k in range(GCH // 16):
                S['idx'][r, pl.ds(k * 16, 16)] = nidx[r, pl.ds(k * 16, 16)]
        pltpu.async_copy(feat_hbm.at[S['idx'].at[0]], S['rows'], S['gsem'])
        pltpu.async_copy(at_hbm.at[S['idx'].at[0]], S['ats'], S['gsem'])
        pltpu.async_copy(at_hbm.at[S['idx'].at[1]], S['atd'], S['gsem'])

    def drain_idx(nidx):
        pltpu.make_async_copy(ei_hbm.at[:, pl.ds(0, GCH)], nidx,
                              isems[id(nidx)]).wait()

    def proc(S):
        pltpu.make_async_copy(feat_hbm.at[pl.ds(0, GCH)], S['rows'], S['gsem']).wait()
        pltpu.make_async_copy(at_hbm.at[pl.ds(0, GCH)], S['ats'], S['gsem']).wait()
        pltpu.make_async_copy(at_hbm.at[pl.ds(0, GCH)], S['atd'], S['gsem']).wait()
        rws, ats_, atd_, exv_ = S['rows'], S['ats'], S['atd'], S['exv']

        def ebody(i2, _):
            exs = []
            for u in range(4):
                i = i2 * 4 + u
                z = (_dyn_gather(ats_[i, :], pidx_l)
                     + _dyn_gather(atd_[i, :], pidx_r))
                z = jnp.maximum(z, 0.2 * z)
                ex = jnp.exp(z - mvec)
                exv_[i, :] = ex
                exs.append(ex)
            for u in range(4):
                i = i2 * 4 + u
                for j in range(D // 16):
                    m = _dyn_gather(exs[u], hsplat[j // 2])
                    rws[i, pl.ds(j * 16, 16)] = rws[i, pl.ds(j * 16, 16)] * m
            return 0
        lax.fori_loop(0, GCH // 4, ebody, 0)

        pltpu.async_copy(S['rows'], snum.at[S['idx'].at[1]], S['ssem'], add=True)
        pltpu.async_copy(S['exv'], sden.at[S['idx'].at[1]], S['ssem'], add=True)

    def drain_scatters(S):
        pltpu.make_async_copy(feat_hbm.at[pl.ds(0, GCH)], S['rows'], S['ssem']).wait()
        pltpu.make_async_copy(at_hbm.at[pl.ds(0, GCH)], S['exv'], S['ssem']).wait()

    fetch_idx(nidx0, 0)
    fetch_idx(nidx1, 1)
    drain_idx(nidx0)
    load(SL[0], nidx0)
    drain_idx(nidx1)
    load(SL[1], nidx1)
    fetch_idx(nidx0, 2)
    fetch_idx(nidx1, 3)

    def body(s, _):
        it0 = 2 * s
        it1 = 2 * s + 1
        proc(SL[0])

        @pl.when(it1 < NI)
        def _():
            proc(SL[1])

        drain_scatters(SL[0])

        @pl.when(it0 + 2 < NI)
        def _():
            drain_idx(nidx0)
            load(SL[0], nidx0)

        @pl.when(it0 + 4 < NI)
        def _():
            fetch_idx(nidx0, it0 + 4)

        @pl.when(it1 < NI)
        def _():
            drain_scatters(SL[1])

        @pl.when(it1 + 2 < NI)
        def _():
            drain_idx(nidx1)
            load(SL[1], nidx1)

        @pl.when(it1 + 4 < NI)
        def _():
            fetch_idx(nidx1, it1 + 4)
        return 0
    lax.fori_loop(0, (NI + 1) // 2, body, 0)

    plsc.subcore_barrier()

    def _flush(off):
        pltpu.sync_copy(snum.at[pl.ds(off, GCH)],
                        num_hbm.at[cid, pl.ds(off, GCH)])
        pltpu.sync_copy(sden.at[pl.ds(off, GCH)],
                        den_hbm.at[cid, pl.ds(off, GCH)])
    _tile_rows(_flush)


def _sc_gat(feat, at, ei, mv):
    mesh = plsc.VectorSubcoreMesh(core_axis_name="c", subcore_axis_name="s")
    slot = [
        pltpu.VMEM((2, GCH), jnp.int32),
        pltpu.VMEM((GCH, D), jnp.float32),
        pltpu.VMEM((GCH, 16), jnp.float32),
        pltpu.VMEM((GCH, 16), jnp.float32),
        pltpu.VMEM((GCH, 16), jnp.float32),
    ]
    f = pl.kernel(
        _sc_gat_body,
        compiler_params=pltpu.CompilerParams(use_tc_tiling_on_sc=False),
        out_type=[
            jax.ShapeDtypeStruct((NC, N, D), jnp.float32),
            jax.ShapeDtypeStruct((NC, N, 16), jnp.float32),
        ],
        mesh=mesh,
        scratch_types=slot + slot + [
            pltpu.VMEM((2, GCH), jnp.int32),
            pltpu.VMEM((2, GCH), jnp.int32),
            pltpu.VMEM((16,), jnp.float32),
            pltpu.VMEM_SHARED((N, D), jnp.float32),
            pltpu.VMEM_SHARED((N, 16), jnp.float32),
            pltpu.SemaphoreType.DMA,
            pltpu.SemaphoreType.DMA,
            pltpu.SemaphoreType.DMA,
            pltpu.SemaphoreType.DMA,
            pltpu.SemaphoreType.DMA,
            pltpu.SemaphoreType.DMA,
        ],
    )
    return f(feat, at, ei, mv)


def _finish_body(with_res, *refs):
    if with_res:
        (n0_ref, n1_ref, d0_ref, d1_ref, b_ref, s_ref, res_ref, out_ref) = refs
    else:
        (n0_ref, n1_ref, d0_ref, d1_ref, b_ref, s_ref, out_ref) = refs
        res_ref = None
    o = None
    for r, (nm, dn) in enumerate(((n0_ref, d0_ref), (n1_ref, d1_ref))):
        numr = nm[0] + nm[1]
        denr = dn[0] + dn[1]
        inv = 1.0 / jnp.where(denr > 0, denr, 1.0)
        inv128 = jnp.dot(inv, s_ref[...], preferred_element_type=jnp.float32)
        v = numr * inv128 + b_ref[r][None, :]
        v = jnp.where(v > 0, v, 0.1 * v)
        o = v if o is None else o + v
    o = 0.5 * o
    if with_res:
        o = o + res_ref[...]
    out_ref[...] = o


def _finish(n0, n1, d0, d1, bias, S, res):
    with_res = res is not None
    in_specs = [
        pl.BlockSpec((NC, BLK, D), lambda i: (0, i, 0)),
        pl.BlockSpec((NC, BLK, D), lambda i: (0, i, 0)),
        pl.BlockSpec((NC, BLK, 16), lambda i: (0, i, 0)),
        pl.BlockSpec((NC, BLK, 16), lambda i: (0, i, 0)),
        pl.BlockSpec((2, D), lambda i: (0, 0)),
        pl.BlockSpec((16, D), lambda i: (0, 0)),
    ]
    args = [n0, n1, d0, d1, bias, S]
    if with_res:
        in_specs.append(pl.BlockSpec((BLK, D), lambda i: (i, 0)))
        args.append(res)
    return pl.pallas_call(
        functools.partial(_finish_body, with_res),
        grid=(N // BLK,),
        in_specs=in_specs,
        out_specs=pl.BlockSpec((BLK, D), lambda i: (i, 0)),
        out_shape=jax.ShapeDtypeStruct((N, D), jnp.float32),
    )(*args)


def kernel(x, edge_index_r0, edge_index_r1, W_emb, al_emb, ar_emb, b_emb,
           W_conv, al_conv, ar_conv, b_conv, bn0_g, bn0_b, bn1_g, bn1_b):
    oh_l = jnp.eye(16, dtype=jnp.float32)[:HEADS]
    oh_r = jnp.eye(16, dtype=jnp.float32)[HEADS:2 * HEADS]

    def pack_a(al, ar):
        a = (al[:, :, :, None] * oh_l[None, :, None, :]
             + ar[:, :, :, None] * oh_r[None, :, None, :])
        return a.reshape(2, D, 16)

    A_emb = pack_a(al_emb, ar_emb)
    A_conv = pack_a(al_conv, ar_conv)
    S = jnp.kron(jnp.eye(HEADS, dtype=jnp.float32), jnp.ones((1, 32), jnp.float32))
    S = jnp.concatenate([S, jnp.zeros((12, D), jnp.float32)], axis=0)

    def stage(h, g, b, W, A, bias, res):
        f0, f1, at0, at1, mcol = _prep(h, g, b, W, A)
        m = jnp.maximum(mcol[:, 0:HEADS] + mcol[:, HEADS:2 * HEADS], 0.0)
        mv = jnp.tile(m, (1, 4))
        n0, d0 = _sc_gat(f0, at0, edge_index_r0, mv[0])
        n1, d1 = _sc_gat(f1, at1, edge_index_r1, mv[1])
        return _finish(n0, n1, d0, d1, bias, S, res)

    emb = stage(x, bn0_g, bn0_b, W_emb, A_emb, b_emb, None)
    h = stage(emb, bn1_g, bn1_b, W_conv, A_conv, b_conv, emb)
    h = stage(h, bn1_g, bn1_b, W_conv, A_conv, b_conv, emb)
    return h


# speedup vs baseline: 1.5745x; 1.5745x over previous
import functools

import jax
import jax.numpy as jnp
from jax import lax
from jax.experimental import pallas as pl
from jax.experimental.pallas import tpu as pltpu
from jax.experimental.pallas import tpu_sc as plsc

N = 10000
E = 320000
D = 128
HEADS = 4
NC = 2
NS = 16
NW = NC * NS
CHUNK = 128
NCHUNKS = E // CHUNK
RPT = 640
BLK = 1000


def _prep_body(h_ref, g_ref, b_ref, w_ref, a_ref,
               f0_ref, f1_ref, at0_ref, at1_ref, mcol_ref):
    nb = pl.program_id(0)

    @pl.when(nb == 0)
    def _():
        mcol_ref[...] = jnp.full((2, 16), -1e30, jnp.float32)

    h = h_ref[...]
    mu = jnp.mean(h, axis=1, keepdims=True)
    hc = h - mu
    var = jnp.mean(hc * hc, axis=1, keepdims=True)
    hn = hc * lax.rsqrt(var + 1e-5) * g_ref[...][None, :] + b_ref[...][None, :]

    f_refs = (f0_ref, f1_ref)
    at_refs = (at0_ref, at1_ref)
    for r in range(2):
        feat = jnp.dot(hn, w_ref[r], preferred_element_type=jnp.float32)
        at = jnp.dot(feat, a_ref[r], preferred_element_type=jnp.float32)
        f_refs[r][...] = feat
        at_refs[r][...] = at
        mcol_ref[r, :] = jnp.maximum(mcol_ref[r, :], jnp.max(at, axis=0))


def _prep(h, g, b, W, A):
    return pl.pallas_call(
        _prep_body,
        grid=(N // BLK,),
        in_specs=[
            pl.BlockSpec((BLK, D), lambda i: (i, 0)),
            pl.BlockSpec((D,), lambda i: (0,)),
            pl.BlockSpec((D,), lambda i: (0,)),
            pl.BlockSpec((2, D, D), lambda i: (0, 0, 0)),
            pl.BlockSpec((2, D, 16), lambda i: (0, 0, 0)),
        ],
        out_specs=[
            pl.BlockSpec((BLK, D), lambda i: (i, 0)),
            pl.BlockSpec((BLK, D), lambda i: (i, 0)),
            pl.BlockSpec((BLK, 16), lambda i: (i, 0)),
            pl.BlockSpec((BLK, 16), lambda i: (i, 0)),
            pl.BlockSpec((2, 16), lambda i: (0, 0)),
        ],
        out_shape=[
            jax.ShapeDtypeStruct((N, D), jnp.float32),
            jax.ShapeDtypeStruct((N, D), jnp.float32),
            jax.ShapeDtypeStruct((N, 16), jnp.float32),
            jax.ShapeDtypeStruct((N, 16), jnp.float32),
            jax.ShapeDtypeStruct((2, 16), jnp.float32),
        ],
    )(h, g, b, W, A)


def _dyn_gather(v, idx):
    dnums = lax.GatherDimensionNumbers(
        offset_dims=(), collapsed_slice_dims=(0,), start_index_map=(0,))
    return lax.gather(v, idx[:, None], dnums, (1,),
                      mode=lax.GatherScatterMode.PROMISE_IN_BOUNDS)


GCH = 80
NI = E // GCH // NW


def _sc_gat_body(feat_hbm, at_hbm, ei_hbm, mv_hbm,
                 num_hbm, den_hbm,
                 idx0, rows0, ats0, atd0, exv0,
                 idx1, rows1, ats1, atd1, exv1,
                 nidx0, nidx1,
                 mvv, snum, sden, gsem0, ssem0, gsem1, ssem1, isem0, isem1):
    cid = lax.axis_index("c")
    sid = lax.axis_index("s")
    wid = sid * NC + cid

    SL = [
        dict(idx=idx0, rows=rows0, ats=ats0, atd=atd0, exv=exv0,
             gsem=gsem0, ssem=ssem0),
        dict(idx=idx1, rows=rows1, ats=ats1, atd=atd1, exv=exv1,
             gsem=gsem1, ssem=ssem1),
    ]
    rows = rows0
    exv = exv0

    zero16 = jnp.zeros((16,), jnp.float32)

    def zbody(i, _):
        for j in range(D // 16):
            rows[i, pl.ds(j * 16, 16)] = zero16
        exv[i, :] = zero16
        return 0
    lax.fori_loop(0, GCH, zbody, 0)

    base = sid * RPT

    def _tile_rows(fn):
        @pl.when(sid < NS - 1)
        def _():
            for k in range(RPT // GCH):
                fn(base + k * GCH)

        @pl.when(sid == NS - 1)
        def _():
            for k in range((N - (NS - 1) * RPT) // GCH):
                fn(base + k * GCH)

    def _zero(off):
        pltpu.sync_copy(rows.at[pl.ds(0, GCH)], snum.at[pl.ds(off, GCH)])
        pltpu.sync_copy(exv.at[pl.ds(0, GCH)], sden.at[pl.ds(off, GCH)])
    _tile_rows(_zero)

    pltpu.sync_copy(mv_hbm, mvv)
    plsc.subcore_barrier()

    mvec = mvv[:]
    lane = lax.iota(jnp.int32, 16)
    pidx_l = lane % 4
    pidx_r = pidx_l + 4
    hsplat = [lane * 0 + h for h in range(HEADS)]

    isems = {id(nidx0): isem0, id(nidx1): isem1}

    def fetch_idx(nidx, it):
        base_e = (wid + it * NW) * GCH
        pltpu.async_copy(ei_hbm.at[:, pl.ds(base_e, GCH)], nidx, isems[id(nidx)])

    def load(S, nidx):
        for r in range(2):
            for k in range(GCH // 16):
                S['idx'][r, pl.ds(k * 16, 16)] = nidx[r, pl.ds(k * 16, 16)]
        pltpu.async_copy(feat_hbm.at[S['idx'].at[0]], S['rows'], S['gsem'])
        pltpu.async_copy(at_hbm.at[S['idx'].at[0]], S['ats'], S['gsem'])
        pltpu.async_copy(at_hbm.at[S['idx'].at[1]], S['atd'], S['gsem'])

    def drain_idx(nidx):
        pltpu.make_async_copy(ei_hbm.at[:, pl.ds(0, GCH)], nidx,
                              isems[id(nidx)]).wait()

    def proc(S):
        pltpu.make_async_copy(feat_hbm.at[pl.ds(0, GCH)], S['rows'], S['gsem']).wait()
        pltpu.make_async_copy(at_hbm.at[pl.ds(0, GCH)], S['ats'], S['gsem']).wait()
        pltpu.make_async_copy(at_hbm.at[pl.ds(0, GCH)], S['atd'], S['gsem']).wait()
        rws, ats_, atd_, exv_ = S['rows'], S['ats'], S['atd'], S['exv']

        def ebody(i2, _):
            exs = []
            for u in range(2):
                i = i2 * 2 + u
                z = (_dyn_gather(ats_[i, :], pidx_l)
                     + _dyn_gather(atd_[i, :], pidx_r))
                z = jnp.maximum(z, 0.2 * z)
                ex = jnp.exp(z - mvec)
                exv_[i, :] = ex
                exs.append(ex)
            for u in range(2):
                i = i2 * 2 + u
                for j in range(D // 16):
                    m = _dyn_gather(exs[u], hsplat[j // 2])
                    rws[i, pl.ds(j * 16, 16)] = rws[i, pl.ds(j * 16, 16)] * m
            return 0
        lax.fori_loop(0, GCH // 2, ebody, 0)

        pltpu.async_copy(S['rows'], snum.at[S['idx'].at[1]], S['ssem'], add=True)
        pltpu.async_copy(S['exv'], sden.at[S['idx'].at[1]], S['ssem'], add=True)

    def drain_scatters(S):
        pltpu.make_async_copy(feat_hbm.at[pl.ds(0, GCH)], S['rows'], S['ssem']).wait()
        pltpu.make_async_copy(at_hbm.at[pl.ds(0, GCH)], S['exv'], S['ssem']).wait()

    fetch_idx(nidx0, 0)
    fetch_idx(nidx1, 1)
    drain_idx(nidx0)
    load(SL[0], nidx0)
    drain_idx(nidx1)
    load(SL[1], nidx1)
    fetch_idx(nidx0, 2)
    fetch_idx(nidx1, 3)

    def body(s, _):
        it0 = 2 * s
        it1 = 2 * s + 1
        proc(SL[0])

        @pl.when(it1 < NI)
        def _():
            proc(SL[1])

        drain_scatters(SL[0])

        @pl.when(it0 + 2 < NI)
        def _():
            drain_idx(nidx0)
            load(SL[0], nidx0)

        @pl.when(it0 + 4 < NI)
        def _():
            fetch_idx(nidx0, it0 + 4)

        @pl.when(it1 < NI)
        def _():
            drain_scatters(SL[1])

        @pl.when(it1 + 2 < NI)
        def _():
            drain_idx(nidx1)
            load(SL[1], nidx1)

        @pl.when(it1 + 4 < NI)
        def _():
            fetch_idx(nidx1, it1 + 4)
        return 0
    lax.fori_loop(0, (NI + 1) // 2, body, 0)

    plsc.subcore_barrier()

    def _flush(off):
        pltpu.sync_copy(snum.at[pl.ds(off, GCH)],
                        num_hbm.at[cid, pl.ds(off, GCH)])
        pltpu.sync_copy(sden.at[pl.ds(off, GCH)],
                        den_hbm.at[cid, pl.ds(off, GCH)])
    _tile_rows(_flush)


def _sc_gat(feat, at, ei, mv):
    mesh = plsc.VectorSubcoreMesh(core_axis_name="c", subcore_axis_name="s")
    slot = [
        pltpu.VMEM((2, GCH), jnp.int32),
        pltpu.VMEM((GCH, D), jnp.float32),
        pltpu.VMEM((GCH, 16), jnp.float32),
        pltpu.VMEM((GCH, 16), jnp.float32),
        pltpu.VMEM((GCH, 16), jnp.float32),
    ]
    f = pl.kernel(
        _sc_gat_body,
        compiler_params=pltpu.CompilerParams(use_tc_tiling_on_sc=False),
        out_type=[
            jax.ShapeDtypeStruct((NC, N, D), jnp.float32),
            jax.ShapeDtypeStruct((NC, N, 16), jnp.float32),
        ],
        mesh=mesh,
        scratch_types=slot + slot + [
            pltpu.VMEM((2, GCH), jnp.int32),
            pltpu.VMEM((2, GCH), jnp.int32),
            pltpu.VMEM((16,), jnp.float32),
            pltpu.VMEM_SHARED((N, D), jnp.float32),
            pltpu.VMEM_SHARED((N, 16), jnp.float32),
            pltpu.SemaphoreType.DMA,
            pltpu.SemaphoreType.DMA,
            pltpu.SemaphoreType.DMA,
            pltpu.SemaphoreType.DMA,
            pltpu.SemaphoreType.DMA,
            pltpu.SemaphoreType.DMA,
        ],
    )
    return f(feat, at, ei, mv)


def _finish_body(with_res, *refs):
    if with_res:
        (n0_ref, n1_ref, d0_ref, d1_ref, b_ref, s_ref, res_ref, out_ref) = refs
    else:
        (n0_ref, n1_ref, d0_ref, d1_ref, b_ref, s_ref, out_ref) = refs
        res_ref = None
    o = None
    for r, (nm, dn) in enumerate(((n0_ref, d0_ref), (n1_ref, d1_ref))):
        numr = nm[0] + nm[1]
        denr = dn[0] + dn[1]
        inv = 1.0 / jnp.where(denr > 0, denr, 1.0)
        inv128 = jnp.dot(inv, s_ref[...], preferred_element_type=jnp.float32)
        v = numr * inv128 + b_ref[r][None, :]
        v = jnp.where(v > 0, v, 0.1 * v)
        o = v if o is None else o + v
    o = 0.5 * o
    if with_res:
        o = o + res_ref[...]
    out_ref[...] = o


def _finish(n0, n1, d0, d1, bias, S, res):
    with_res = res is not None
    in_specs = [
        pl.BlockSpec((NC, BLK, D), lambda i: (0, i, 0)),
        pl.BlockSpec((NC, BLK, D), lambda i: (0, i, 0)),
        pl.BlockSpec((NC, BLK, 16), lambda i: (0, i, 0)),
        pl.BlockSpec((NC, BLK, 16), lambda i: (0, i, 0)),
        pl.BlockSpec((2, D), lambda i: (0, 0)),
        pl.BlockSpec((16, D), lambda i: (0, 0)),
    ]
    args = [n0, n1, d0, d1, bias, S]
    if with_res:
        in_specs.append(pl.BlockSpec((BLK, D), lambda i: (i, 0)))
        args.append(res)
    return pl.pallas_call(
        functools.partial(_finish_body, with_res),
        grid=(N // BLK,),
        in_specs=in_specs,
        out_specs=pl.BlockSpec((BLK, D), lambda i: (i, 0)),
        out_shape=jax.ShapeDtypeStruct((N, D), jnp.float32),
    )(*args)


def kernel(x, edge_index_r0, edge_index_r1, W_emb, al_emb, ar_emb, b_emb,
           W_conv, al_conv, ar_conv, b_conv, bn0_g, bn0_b, bn1_g, bn1_b):
    oh_l = jnp.eye(16, dtype=jnp.float32)[:HEADS]
    oh_r = jnp.eye(16, dtype=jnp.float32)[HEADS:2 * HEADS]

    def pack_a(al, ar):
        a = (al[:, :, :, None] * oh_l[None, :, None, :]
             + ar[:, :, :, None] * oh_r[None, :, None, :])
        return a.reshape(2, D, 16)

    A_emb = pack_a(al_emb, ar_emb)
    A_conv = pack_a(al_conv, ar_conv)
    S = jnp.kron(jnp.eye(HEADS, dtype=jnp.float32), jnp.ones((1, 32), jnp.float32))
    S = jnp.concatenate([S, jnp.zeros((12, D), jnp.float32)], axis=0)

    def stage(h, g, b, W, A, bias, res):
        f0, f1, at0, at1, mcol = _prep(h, g, b, W, A)
        m = jnp.maximum(mcol[:, 0:HEADS] + mcol[:, HEADS:2 * HEADS], 0.0)
        mv = jnp.tile(m, (1, 4))
        n0, d0 = _sc_gat(f0, at0, edge_index_r0, mv[0])
        n1, d1 = _sc_gat(f1, at1, edge_index_r1, mv[1])
        return _finish(n0, n1, d0, d1, bias, S, res)

    emb = stage(x, bn0_g, bn0_b, W_emb, A_emb, b_emb, None)
    h = stage(emb, bn1_g, bn1_b, W_conv, A_conv, b_conv, emb)
    h = stage(h, bn1_g, bn1_b, W_conv, A_conv, b_conv, emb)
    return h
